# Initial kernel scaffold; baseline (speedup 1.0000x reference)
import jax
import jax.numpy as jnp
from jax.experimental import pallas as pl


def kernel(x, edge_index, W1, b1, W2, b2, g1, be1, W3, b3, W4, b4, g2, be2, W5, b5, W6, b6):
    raise NotImplementedError("write your pallas kernel here")


# speedup vs baseline: 3.5255x; 3.5255x over previous
import functools

import jax
import jax.numpy as jnp
from jax import lax
from jax.experimental import pallas as pl
from jax.experimental.pallas import tpu as pltpu
from jax.experimental.pallas import tpu_sc as plsc

_NC = 2
_NS = 16
_NW = _NC * _NS
_K = 80
_R = 400
_EPS = 1e-5


def _seg_sum_sc(hs, src3, dst3, n):
    nk = len(hs)
    c = hs[0].shape[1]
    steps = src3.shape[1]
    npad = -(-n // (_NS * 8)) * (_NS * 8)
    rpt = npad // _NS
    zeros = jnp.zeros((rpt, c), jnp.float32)
    mesh = plsc.VectorSubcoreMesh(core_axis_name="c", subcore_axis_name="s")

    @functools.partial(
        pl.kernel,
        out_type=jax.ShapeDtypeStruct((_NC * nk, npad, c), jnp.float32),
        mesh=mesh,
        scratch_types=[
            pltpu.VMEM((steps, _K), jnp.int32),
            pltpu.VMEM((steps, _K), jnp.int32),
            pltpu.VMEM((_K, c), jnp.float32),
            pltpu.VMEM_SHARED((npad, c), jnp.float32),
            pltpu.SemaphoreType.DMA,
        ],
    )
    def seg(src_hbm, dst_hbm, zero_hbm, *rest):
        h_refs = rest[:nk]
        out = rest[nk]
        srcv, dstv, rows, acc, sem = rest[nk + 1:]
        cid = lax.axis_index("c")
        sid = lax.axis_index("s")
        wid = sid * _NC + cid
        row0 = sid * rpt
        pltpu.sync_copy(src_hbm.at[wid], srcv)
        pltpu.sync_copy(dst_hbm.at[wid], dstv)
        for k in range(nk):
            pltpu.sync_copy(zero_hbm, acc.at[pl.ds(row0, rpt)])
            plsc.subcore_barrier()

            def step(i, carry, k=k):
                pltpu.async_copy(h_refs[k].at[srcv.at[i]], rows, sem).wait()
                pltpu.sync_copy(rows, acc.at[dstv.at[i]], add=True)
                return carry

            lax.fori_loop(0, steps, step, 0)
            plsc.subcore_barrier()
            pltpu.sync_copy(acc.at[pl.ds(row0, rpt)],
                            out.at[cid * nk + k, pl.ds(row0, rpt)])
            plsc.subcore_barrier()

    return seg(src3, dst3, zeros, *hs)


def _mlp_body(nk, n, x_refs, p_ref, w1_ref, b1_ref, w2_ref, b2_ref,
              h_ref, s_ref, q_ref):
    i = pl.program_id(0)
    h0 = jnp.concatenate(
        [x_refs[k][...] + p_ref[k] + p_ref[nk + k] for k in range(nk)],
        axis=1) if nk > 1 else x_refs[0][...] + p_ref[0] + p_ref[1]
    a = jnp.maximum(
        jnp.dot(h0, w1_ref[...], preferred_element_type=jnp.float32)
        + b1_ref[...], 0.0)
    h = (jnp.dot(a, w2_ref[...], preferred_element_type=jnp.float32)
         + b2_ref[...])
    h_ref[...] = h
    s = jnp.sum(h, axis=0, keepdims=True)
    q = jnp.sum(h * h, axis=0, keepdims=True)

    @pl.when(i == 0)
    def _():
        s_ref[...] = s
        q_ref[...] = q

    @pl.when(i > 0)
    def _():
        s_ref[...] += s
        q_ref[...] += q


def _mlp_call(xs, p, w1, b1, w2, b2):
    nk = len(xs)
    n = xs[0].shape[0]
    d = 128 * nk
    h = w1.shape[1]
    grid = (n // _R,)

    def body(*refs):
        _mlp_body(nk, n, refs[:nk], refs[nk], refs[nk + 1], refs[nk + 2],
                  refs[nk + 3], refs[nk + 4], refs[nk + 5], refs[nk + 6],
                  refs[nk + 7])

    return pl.pallas_call(
        body,
        grid=grid,
        in_specs=[pl.BlockSpec((_R, 128), lambda i: (i, 0))] * nk + [
            pl.BlockSpec((2 * nk, _R, 128), lambda i: (0, i, 0)),
            pl.BlockSpec((d, h), lambda i: (0, 0)),
            pl.BlockSpec((1, h), lambda i: (0, 0)),
            pl.BlockSpec((h, h), lambda i: (0, 0)),
            pl.BlockSpec((1, h), lambda i: (0, 0)),
        ],
        out_specs=[
            pl.BlockSpec((_R, h), lambda i: (i, 0)),
            pl.BlockSpec((1, h), lambda i: (0, 0)),
            pl.BlockSpec((1, h), lambda i: (0, 0)),
        ],
        out_shape=[
            jax.ShapeDtypeStruct((n, h), jnp.float32),
            jax.ShapeDtypeStruct((1, h), jnp.float32),
            jax.ShapeDtypeStruct((1, h), jnp.float32),
        ],
    )(*xs, p, w1, b1.reshape(1, h), w2, b2.reshape(1, h))


def _bn_chunk_call(hpre, s, q, g, be):
    n, h = hpre.shape
    nk = h // 128
    grid = (n // _R,)

    def body(h_ref, s_ref, q_ref, g_ref, be_ref, *outs):
        m = s_ref[...] / n
        v = q_ref[...] / n - m * m
        inv = lax.rsqrt(v + _EPS) * g_ref[...]
        hn = jnp.maximum((h_ref[...] - m) * inv + be_ref[...], 0.0)
        for k in range(nk):
            outs[k][...] = hn[:, k * 128:(k + 1) * 128]

    return pl.pallas_call(
        body,
        grid=grid,
        in_specs=[
            pl.BlockSpec((_R, h), lambda i: (i, 0)),
            pl.BlockSpec((1, h), lambda i: (0, 0)),
            pl.BlockSpec((1, h), lambda i: (0, 0)),
            pl.BlockSpec((1, h), lambda i: (0, 0)),
            pl.BlockSpec((1, h), lambda i: (0, 0)),
        ],
        out_specs=[pl.BlockSpec((_R, 128), lambda i: (i, 0))] * nk,
        out_shape=[jax.ShapeDtypeStruct((n, 128), jnp.float32)] * nk,
    )(hpre, s, q, g.reshape(1, h), be.reshape(1, h))


def _head_call(hpre, s, q, g, be, w5, b5, w6, b6):
    n, h = hpre.shape
    fc = w5.shape[1]
    out = w6.shape[1]
    grid = (n // _R,)

    def body(h_ref, s_ref, q_ref, g_ref, be_ref, w5_ref, b5_ref, w6_ref,
             b6_ref, o_ref):
        m = s_ref[...] / n
        v = q_ref[...] / n - m * m
        inv = lax.rsqrt(v + _EPS) * g_ref[...]
        hn = jnp.maximum((h_ref[...] - m) * inv + be_ref[...], 0.0)
        a = jnp.maximum(
            jnp.dot(hn, w5_ref[...], preferred_element_type=jnp.float32)
            + b5_ref[...], 0.0)
        o_ref[...] = (jnp.dot(a, w6_ref[...],
                              preferred_element_type=jnp.float32)
                      + b6_ref[...])

    return pl.pallas_call(
        body,
        grid=grid,
        in_specs=[
            pl.BlockSpec((_R, h), lambda i: (i, 0)),
            pl.BlockSpec((1, h), lambda i: (0, 0)),
            pl.BlockSpec((1, h), lambda i: (0, 0)),
            pl.BlockSpec((1, h), lambda i: (0, 0)),
            pl.BlockSpec((1, h), lambda i: (0, 0)),
            pl.BlockSpec((h, fc), lambda i: (0, 0)),
            pl.BlockSpec((1, fc), lambda i: (0, 0)),
            pl.BlockSpec((fc, out), lambda i: (0, 0)),
            pl.BlockSpec((1, out), lambda i: (0, 0)),
        ],
        out_specs=pl.BlockSpec((_R, out), lambda i: (i, 0)),
        out_shape=jax.ShapeDtypeStruct((n, out), jnp.float32),
    )(hpre, s, q, g.reshape(1, h), be.reshape(1, h), w5, b5.reshape(1, fc),
      w6, b6.reshape(1, out))


def kernel(x, edge_index, W1, b1, W2, b2, g1, be1, W3, b3, W4, b4, g2, be2,
           W5, b5, W6, b6):
    n = x.shape[0]
    e = edge_index.shape[1]
    steps = e // _NW // _K
    src3 = edge_index[0].astype(jnp.int32).reshape(_NW, steps, _K)
    dst3 = edge_index[1].astype(jnp.int32).reshape(_NW, steps, _K)

    p1 = _seg_sum_sc([x], src3, dst3, n)
    h1pre, s1, q1 = _mlp_call([x], p1, W1, b1, W2, b2)
    h1c = _bn_chunk_call(h1pre, s1, q1, g1, be1)

    p2 = _seg_sum_sc(h1c, src3, dst3, n)
    h2pre, s2, q2 = _mlp_call(h1c, p2, W3, b3, W4, b4)

    return _head_call(h2pre, s2, q2, g2, be2, W5, b5, W6, b6)
